# baseline (speedup 1.0000x reference)
import functools

import jax
import jax.numpy as jnp
from jax.experimental import pallas as pl
from jax.experimental.pallas import tpu as pltpu

B, P, C = 32, 8732, 21
TP = 4368
NP = 2


def _isd_kernel(lam_ref, conf_ref, confsh_ref, confi_ref, loc_ref, locsh_ref,
                loci_ref, out_ref):
    b = pl.program_id(0)
    ip = pl.program_id(1)
    lam = lam_ref[0]

    conf = conf_ref[0]
    ctemp = confsh_ref[0]
    cinterp = confi_ref[0]

    rows = jax.lax.broadcasted_iota(jnp.int32, (TP, 1), 0)
    valid = (ip * TP + rows) < P
    validf = valid.astype(jnp.float32)

    conf = jnp.where(valid, conf, 1.0)
    ctemp = jnp.where(valid, ctemp, 1.0)
    cinterp = jnp.where(valid, cinterp, 1.0)

    lmf = (jnp.max(conf[:, 1:], axis=1, keepdims=True)
           > conf[:, 0:1]).astype(jnp.float32) * validf
    rmf = (jnp.max(ctemp[:, 1:], axis=1, keepdims=True)
           > ctemp[:, 0:1]).astype(jnp.float32) * validf
    only_l = lmf * (1.0 - rmf)
    only_r = rmf * (1.0 - lmf)
    inter = lmf * rmf

    mixed = lam * conf + (1.0 - lam) * ctemp + 1e-07
    interp = cinterp + 1e-07
    log_i = jnp.log(interp)
    log_m = jnp.log(mixed)
    kl_ab = jnp.sum((interp - mixed) * (log_i - log_m), axis=1, keepdims=True)

    ce = conf + 1e-07
    cte = ctemp + 1e-07
    kl_l = jnp.sum(ce * (jnp.log(ce) - log_i), axis=1, keepdims=True)
    kl_r = jnp.sum(cte * (jnp.log(cte) - log_i), axis=1, keepdims=True)

    loc = jnp.where(valid, loc_ref[0], 0.0)
    locsh = jnp.where(valid, locsh_ref[0], 0.0)
    loci = jnp.where(valid, loci_ref[0], 0.0)
    sse_l = jnp.sum(((loci - loc) ** 2) * only_l, axis=0)
    sse_r = jnp.sum(((loci - locsh) ** 2) * only_r, axis=0)

    @pl.when((b == 0) & (ip == 0))
    def _init():
        for i in range(14):
            out_ref[i] = 0.0

    out_ref[0] += jnp.sum(kl_ab * inter)
    out_ref[1] += jnp.sum(inter)
    out_ref[2] += jnp.sum(kl_l * only_l)
    out_ref[3] += jnp.sum(only_l)
    out_ref[4] += jnp.sum(kl_r * only_r)
    out_ref[5] += jnp.sum(only_r)
    for i in range(4):
        out_ref[6 + i] += sse_l[i]
        out_ref[10 + i] += sse_r[i]


@jax.jit
def _isd_loss(lam, conf, conf_shuffle, conf_interpolation, loc, loc_shuffle,
              loc_interpolation):
    half = B // 2
    cspec = pl.BlockSpec((1, TP, C), lambda b, ip: (b, ip, 0))
    cspec_swap = pl.BlockSpec((1, TP, C), lambda b, ip: ((b + half) % B, ip, 0))
    lspec = pl.BlockSpec((1, TP, 4), lambda b, ip: (b, ip, 0))
    lspec_swap = pl.BlockSpec((1, TP, 4), lambda b, ip: ((b + half) % B, ip, 0))

    sums = pl.pallas_call(
        _isd_kernel,
        grid=(B, NP),
        in_specs=[
            pl.BlockSpec(memory_space=pltpu.SMEM),
            cspec, cspec_swap, cspec, lspec, lspec_swap, lspec,
        ],
        out_specs=pl.BlockSpec(memory_space=pltpu.SMEM),
        out_shape=jax.ShapeDtypeStruct((14,), jnp.float32),
        compiler_params=pltpu.CompilerParams(
            dimension_semantics=("arbitrary", "arbitrary"),
        ),
    )(lam.reshape(1).astype(jnp.float32), conf, conf_shuffle,
      conf_interpolation, loc, loc_shuffle, loc_interpolation)

    def mmean(s, c):
        return jnp.where(c > 0, s / jnp.maximum(c, 1.0), 0.0)

    interp_loss = mmean(sums[0], sums[1]) / 2.0
    fixmatch = (
        mmean(sums[2], sums[3])
        + (mmean(sums[6], sums[3]) + mmean(sums[7], sums[3])
           + mmean(sums[8], sums[3]) + mmean(sums[9], sums[3])) / 4.0
        + mmean(sums[4], sums[5])
        + (mmean(sums[10], sums[5]) + mmean(sums[11], sums[5])
           + mmean(sums[12], sums[5]) + mmean(sums[13], sums[5])) / 4.0
    )
    return interp_loss, fixmatch


def kernel(lam, conf, conf_flip, loc, loc_flip, conf_shuffle,
           conf_interpolation, loc_shuffle, loc_interpolation):
    del conf_flip, loc_flip
    return _isd_loss(lam, conf, conf_shuffle, conf_interpolation, loc,
                     loc_shuffle, loc_interpolation)


# speedup vs baseline: 1.2238x; 1.2238x over previous
import jax
import jax.numpy as jnp
from jax.experimental import pallas as pl
from jax.experimental.pallas import tpu as pltpu

B, P, C = 32, 8732, 21
TP = 4480
NP = 2


def _tpose(x, n):
    e = jnp.eye(n, dtype=jnp.float32)
    return jax.lax.dot_general(e, x, (((1,), (1,)), ((), ())),
                               preferred_element_type=jnp.float32)


def _isd_kernel(lam_ref, conf_ref, confsh_ref, confi_ref, loc_ref, locsh_ref,
                loci_ref, out_ref, acc_ref):
    b = pl.program_id(0)
    ip = pl.program_id(1)
    lam = lam_ref[0]

    lanes = jax.lax.broadcasted_iota(jnp.int32, (1, TP), 1)
    valid = (ip * TP + lanes) < P

    tc = _tpose(conf_ref[0], C)
    tt = _tpose(confsh_ref[0], C)
    ti = _tpose(confi_ref[0], C)
    tc = jnp.where(valid, tc, 1.0)
    tt = jnp.where(valid, tt, 1.0)
    ti = jnp.where(valid, ti, 1.0)

    lmf = (jnp.max(tc[1:], axis=0, keepdims=True)
           > tc[0:1]).astype(jnp.float32)
    rmf = (jnp.max(tt[1:], axis=0, keepdims=True)
           > tt[0:1]).astype(jnp.float32)
    inter = lmf * rmf
    only_l = lmf - inter
    only_r = rmf - inter

    mixed = lam * tc + (1.0 - lam) * tt + 1e-07
    interp = ti + 1e-07
    kl_ab = jnp.sum((interp - mixed) * jnp.log(interp / mixed),
                    axis=0, keepdims=True)
    ce = tc + 1e-07
    cte = tt + 1e-07
    kl_l = jnp.sum(ce * jnp.log(ce / interp), axis=0, keepdims=True)
    kl_r = jnp.sum(cte * jnp.log(cte / interp), axis=0, keepdims=True)

    tl = jnp.where(valid, _tpose(loc_ref[0], 4), 0.0)
    tls = jnp.where(valid, _tpose(locsh_ref[0], 4), 0.0)
    tli = jnp.where(valid, _tpose(loci_ref[0], 4), 0.0)
    se_l = jnp.sum((tli - tl) ** 2, axis=0, keepdims=True)
    se_r = jnp.sum((tli - tls) ** 2, axis=0, keepdims=True)

    @pl.when((b == 0) & (ip == 0))
    def _init():
        acc_ref[...] = jnp.zeros((8, TP), jnp.float32)

    acc_ref[0:1] += kl_ab * inter
    acc_ref[1:2] += inter
    acc_ref[2:3] += kl_l * only_l
    acc_ref[3:4] += only_l
    acc_ref[4:5] += kl_r * only_r
    acc_ref[5:6] += only_r
    acc_ref[6:7] += se_l * only_l
    acc_ref[7:8] += se_r * only_r

    @pl.when((b == B - 1) & (ip == NP - 1))
    def _fin():
        for j in range(8):
            out_ref[j] = jnp.sum(acc_ref[j, :])


@jax.jit
def _isd_loss(lam, conf, conf_shuffle, conf_interpolation, loc, loc_shuffle,
              loc_interpolation):
    half = B // 2
    cspec = pl.BlockSpec((1, TP, C), lambda b, ip: (b, ip, 0))
    cspec_swap = pl.BlockSpec((1, TP, C), lambda b, ip: ((b + half) % B, ip, 0))
    lspec = pl.BlockSpec((1, TP, 4), lambda b, ip: (b, ip, 0))
    lspec_swap = pl.BlockSpec((1, TP, 4), lambda b, ip: ((b + half) % B, ip, 0))

    sums = pl.pallas_call(
        _isd_kernel,
        grid=(B, NP),
        in_specs=[
            pl.BlockSpec(memory_space=pltpu.SMEM),
            cspec, cspec_swap, cspec, lspec, lspec_swap, lspec,
        ],
        out_specs=pl.BlockSpec(memory_space=pltpu.SMEM),
        out_shape=jax.ShapeDtypeStruct((8,), jnp.float32),
        scratch_shapes=[pltpu.VMEM((8, TP), jnp.float32)],
        compiler_params=pltpu.CompilerParams(
            dimension_semantics=("arbitrary", "arbitrary"),
        ),
    )(lam.reshape(1).astype(jnp.float32), conf, conf_shuffle,
      conf_interpolation, loc, loc_shuffle, loc_interpolation)

    def mmean(s, c):
        return jnp.where(c > 0, s / jnp.maximum(c, 1.0), 0.0)

    interp_loss = mmean(sums[0], sums[1]) / 2.0
    fixmatch = (mmean(sums[2], sums[3]) + mmean(sums[6], sums[3]) / 4.0
                + mmean(sums[4], sums[5]) + mmean(sums[7], sums[5]) / 4.0)
    return interp_loss, fixmatch


def kernel(lam, conf, conf_flip, loc, loc_flip, conf_shuffle,
           conf_interpolation, loc_shuffle, loc_interpolation):
    del conf_flip, loc_flip
    return _isd_loss(lam, conf, conf_shuffle, conf_interpolation, loc,
                     loc_shuffle, loc_interpolation)


# speedup vs baseline: 5.0615x; 4.1359x over previous
import jax
import jax.numpy as jnp
from jax.experimental import pallas as pl
from jax.experimental.pallas import tpu as pltpu

B, P, C = 32, 8732, 21
TP = 4480
NP = 2


def _isd_kernel(lam_ref, conf_ref, confsh_ref, confi_ref, loc_ref, locsh_ref,
                loci_ref, out_ref, acc_ref):
    b = pl.program_id(0)
    ip = pl.program_id(1)
    lam = lam_ref[0]

    lanes = jax.lax.broadcasted_iota(jnp.int32, (1, TP), 1)
    valid = (ip * TP + lanes) < P

    tc = conf_ref[0]
    tt = confsh_ref[0]
    ti = confi_ref[0]
    tc = jnp.where(valid, tc, 1.0)
    tt = jnp.where(valid, tt, 1.0)
    ti = jnp.where(valid, ti, 1.0)

    lmf = (jnp.max(tc[1:], axis=0, keepdims=True)
           > tc[0:1]).astype(jnp.float32)
    rmf = (jnp.max(tt[1:], axis=0, keepdims=True)
           > tt[0:1]).astype(jnp.float32)
    inter = lmf * rmf
    only_l = lmf - inter
    only_r = rmf - inter

    mixed = lam * tc + (1.0 - lam) * tt + 1e-07
    interp = ti + 1e-07
    kl_ab = jnp.sum((interp - mixed) * jnp.log(interp / mixed),
                    axis=0, keepdims=True)
    ce = tc + 1e-07
    cte = tt + 1e-07
    kl_l = jnp.sum(ce * jnp.log(ce / interp), axis=0, keepdims=True)
    kl_r = jnp.sum(cte * jnp.log(cte / interp), axis=0, keepdims=True)

    tl = jnp.where(valid, loc_ref[0], 0.0)
    tls = jnp.where(valid, locsh_ref[0], 0.0)
    tli = jnp.where(valid, loci_ref[0], 0.0)
    se_l = jnp.sum((tli - tl) ** 2, axis=0, keepdims=True)
    se_r = jnp.sum((tli - tls) ** 2, axis=0, keepdims=True)

    @pl.when((b == 0) & (ip == 0))
    def _init():
        acc_ref[...] = jnp.zeros((8, TP), jnp.float32)

    acc_ref[0:1] += kl_ab * inter
    acc_ref[1:2] += inter
    acc_ref[2:3] += kl_l * only_l
    acc_ref[3:4] += only_l
    acc_ref[4:5] += kl_r * only_r
    acc_ref[5:6] += only_r
    acc_ref[6:7] += se_l * only_l
    acc_ref[7:8] += se_r * only_r

    @pl.when((b == B - 1) & (ip == NP - 1))
    def _fin():
        for j in range(8):
            out_ref[j] = jnp.sum(acc_ref[j, :])


@jax.jit
def _isd_loss(lam, conf, conf_shuffle, conf_interpolation, loc, loc_shuffle,
              loc_interpolation):
    half = B // 2
    cspec = pl.BlockSpec((1, C, TP), lambda b, ip: (b, 0, ip))
    cspec_swap = pl.BlockSpec((1, C, TP), lambda b, ip: ((b + half) % B, 0, ip))
    lspec = pl.BlockSpec((1, 4, TP), lambda b, ip: (b, 0, ip))
    lspec_swap = pl.BlockSpec((1, 4, TP), lambda b, ip: ((b + half) % B, 0, ip))

    sums = pl.pallas_call(
        _isd_kernel,
        grid=(B, NP),
        in_specs=[
            pl.BlockSpec(memory_space=pltpu.SMEM),
            cspec, cspec_swap, cspec, lspec, lspec_swap, lspec,
        ],
        out_specs=pl.BlockSpec(memory_space=pltpu.SMEM),
        out_shape=jax.ShapeDtypeStruct((8,), jnp.float32),
        scratch_shapes=[pltpu.VMEM((8, TP), jnp.float32)],
        compiler_params=pltpu.CompilerParams(
            dimension_semantics=("arbitrary", "arbitrary"),
        ),
    )(lam.reshape(1).astype(jnp.float32),
      jnp.swapaxes(conf, 1, 2), jnp.swapaxes(conf_shuffle, 1, 2),
      jnp.swapaxes(conf_interpolation, 1, 2), jnp.swapaxes(loc, 1, 2),
      jnp.swapaxes(loc_shuffle, 1, 2), jnp.swapaxes(loc_interpolation, 1, 2))

    def mmean(s, c):
        return jnp.where(c > 0, s / jnp.maximum(c, 1.0), 0.0)

    interp_loss = mmean(sums[0], sums[1]) / 2.0
    fixmatch = (mmean(sums[2], sums[3]) + mmean(sums[6], sums[3]) / 4.0
                + mmean(sums[4], sums[5]) + mmean(sums[7], sums[5]) / 4.0)
    return interp_loss, fixmatch


def kernel(lam, conf, conf_flip, loc, loc_flip, conf_shuffle,
           conf_interpolation, loc_shuffle, loc_interpolation):
    del conf_flip, loc_flip
    return _isd_loss(lam, conf, conf_shuffle, conf_interpolation, loc,
                     loc_shuffle, loc_interpolation)


# speedup vs baseline: 5.1268x; 1.0129x over previous
import jax
import jax.numpy as jnp
from jax.experimental import pallas as pl
from jax.experimental.pallas import tpu as pltpu

B, P, C = 32, 8732, 21
TP = 4480
NP = 2


def _isd_kernel(lam_ref, conf_ref, confsh_ref, confi_ref, loc_ref, locsh_ref,
                loci_ref, out_ref, acc_ref):
    b = pl.program_id(0)
    ip = pl.program_id(1)
    lam = lam_ref[0]

    lanes = jax.lax.broadcasted_iota(jnp.int32, (1, TP), 1)
    valid = (ip * TP + lanes) < P
    validf = valid.astype(jnp.float32)

    tc = conf_ref[0]
    tt = confsh_ref[0]
    ti = confi_ref[0]

    lmf = (jnp.max(tc, axis=0, keepdims=True)
           > tc[0:1]).astype(jnp.float32) * validf
    rmf = (jnp.max(tt, axis=0, keepdims=True)
           > tt[0:1]).astype(jnp.float32) * validf
    inter = lmf * rmf
    only_l = lmf - inter
    only_r = rmf - inter

    mixed = lam * tc + (1.0 - lam) * tt + 1e-07
    interp = ti + 1e-07
    rint = 1.0 / interp
    kl_ab = jnp.sum((interp - mixed) * jnp.log(interp / mixed),
                    axis=0, keepdims=True)
    ce = tc + 1e-07
    cte = tt + 1e-07
    kl_l = jnp.sum(ce * jnp.log(ce * rint), axis=0, keepdims=True)
    kl_r = jnp.sum(cte * jnp.log(cte * rint), axis=0, keepdims=True)
    kl_ab = jnp.where(valid, kl_ab, 0.0)
    kl_l = jnp.where(valid, kl_l, 0.0)
    kl_r = jnp.where(valid, kl_r, 0.0)

    tl = loc_ref[0]
    tls = locsh_ref[0]
    tli = loci_ref[0]
    se_l = jnp.where(valid, jnp.sum((tli - tl) ** 2, axis=0, keepdims=True),
                     0.0)
    se_r = jnp.where(valid, jnp.sum((tli - tls) ** 2, axis=0, keepdims=True),
                     0.0)

    @pl.when((b == 0) & (ip == 0))
    def _init():
        acc_ref[...] = jnp.zeros((8, TP), jnp.float32)

    acc_ref[0:1] += kl_ab * inter
    acc_ref[1:2] += inter
    acc_ref[2:3] += kl_l * only_l
    acc_ref[3:4] += only_l
    acc_ref[4:5] += kl_r * only_r
    acc_ref[5:6] += only_r
    acc_ref[6:7] += se_l * only_l
    acc_ref[7:8] += se_r * only_r

    @pl.when((b == B - 1) & (ip == NP - 1))
    def _fin():
        for j in range(8):
            out_ref[j] = jnp.sum(acc_ref[j, :])


@jax.jit
def _isd_loss(lam, conf, conf_shuffle, conf_interpolation, loc, loc_shuffle,
              loc_interpolation):
    half = B // 2
    cspec = pl.BlockSpec((1, C, TP), lambda b, ip: (b, 0, ip))
    cspec_swap = pl.BlockSpec((1, C, TP), lambda b, ip: ((b + half) % B, 0, ip))
    lspec = pl.BlockSpec((1, 4, TP), lambda b, ip: (b, 0, ip))
    lspec_swap = pl.BlockSpec((1, 4, TP), lambda b, ip: ((b + half) % B, 0, ip))

    sums = pl.pallas_call(
        _isd_kernel,
        grid=(B, NP),
        in_specs=[
            pl.BlockSpec(memory_space=pltpu.SMEM),
            cspec, cspec_swap, cspec, lspec, lspec_swap, lspec,
        ],
        out_specs=pl.BlockSpec(memory_space=pltpu.SMEM),
        out_shape=jax.ShapeDtypeStruct((8,), jnp.float32),
        scratch_shapes=[pltpu.VMEM((8, TP), jnp.float32)],
        compiler_params=pltpu.CompilerParams(
            dimension_semantics=("arbitrary", "arbitrary"),
        ),
    )(lam.reshape(1).astype(jnp.float32),
      jnp.swapaxes(conf, 1, 2), jnp.swapaxes(conf_shuffle, 1, 2),
      jnp.swapaxes(conf_interpolation, 1, 2), jnp.swapaxes(loc, 1, 2),
      jnp.swapaxes(loc_shuffle, 1, 2), jnp.swapaxes(loc_interpolation, 1, 2))

    def mmean(s, c):
        return jnp.where(c > 0, s / jnp.maximum(c, 1.0), 0.0)

    interp_loss = mmean(sums[0], sums[1]) / 2.0
    fixmatch = (mmean(sums[2], sums[3]) + mmean(sums[6], sums[3]) / 4.0
                + mmean(sums[4], sums[5]) + mmean(sums[7], sums[5]) / 4.0)
    return interp_loss, fixmatch


def kernel(lam, conf, conf_flip, loc, loc_flip, conf_shuffle,
           conf_interpolation, loc_shuffle, loc_interpolation):
    del conf_flip, loc_flip
    return _isd_loss(lam, conf, conf_shuffle, conf_interpolation, loc,
                     loc_shuffle, loc_interpolation)


# speedup vs baseline: 5.3742x; 1.0483x over previous
import jax
import jax.numpy as jnp
from jax.experimental import pallas as pl
from jax.experimental.pallas import tpu as pltpu

B, P, C = 32, 8732, 21


def _isd_kernel(lam_ref, conf_ref, confsh_ref, confi_ref, loc_ref, locsh_ref,
                loci_ref, out_ref, acc_ref):
    b = pl.program_id(0)
    lam = lam_ref[0]

    tc = conf_ref[0]
    tt = confsh_ref[0]
    ti = confi_ref[0]

    lmf = (jnp.max(tc, axis=0, keepdims=True)
           > tc[0:1]).astype(jnp.float32)
    rmf = (jnp.max(tt, axis=0, keepdims=True)
           > tt[0:1]).astype(jnp.float32)
    inter = lmf * rmf
    only_l = lmf - inter
    only_r = rmf - inter

    mixed = lam * tc + (1.0 - lam) * tt + 1e-07
    interp = ti + 1e-07
    ce = tc + 1e-07
    cte = tt + 1e-07
    log_i = jnp.log(interp)
    log_m = jnp.log(mixed)
    kl_ab = jnp.sum((interp - mixed) * (log_i - log_m), axis=0, keepdims=True)
    kl_l = jnp.sum(ce * (jnp.log(ce) - log_i), axis=0, keepdims=True)
    kl_r = jnp.sum(cte * (jnp.log(cte) - log_i), axis=0, keepdims=True)

    se_l = jnp.sum((loci_ref[0] - loc_ref[0]) ** 2, axis=0, keepdims=True)
    se_r = jnp.sum((loci_ref[0] - locsh_ref[0]) ** 2, axis=0, keepdims=True)

    @pl.when(b == 0)
    def _init():
        acc_ref[...] = jnp.zeros((8, P), jnp.float32)

    acc_ref[0:1] += kl_ab * inter
    acc_ref[1:2] += inter
    acc_ref[2:3] += kl_l * only_l
    acc_ref[3:4] += only_l
    acc_ref[4:5] += kl_r * only_r
    acc_ref[5:6] += only_r
    acc_ref[6:7] += se_l * only_l
    acc_ref[7:8] += se_r * only_r

    @pl.when(b == B - 1)
    def _fin():
        for j in range(8):
            out_ref[j] = jnp.sum(acc_ref[j, :])


@jax.jit
def _isd_loss(lam, conf, conf_shuffle, conf_interpolation, loc, loc_shuffle,
              loc_interpolation):
    half = B // 2
    cspec = pl.BlockSpec((1, C, P), lambda b: (b, 0, 0))
    cspec_swap = pl.BlockSpec((1, C, P), lambda b: ((b + half) % B, 0, 0))
    lspec = pl.BlockSpec((1, 4, P), lambda b: (b, 0, 0))
    lspec_swap = pl.BlockSpec((1, 4, P), lambda b: ((b + half) % B, 0, 0))

    sums = pl.pallas_call(
        _isd_kernel,
        grid=(B,),
        in_specs=[
            pl.BlockSpec(memory_space=pltpu.SMEM),
            cspec, cspec_swap, cspec, lspec, lspec_swap, lspec,
        ],
        out_specs=pl.BlockSpec(memory_space=pltpu.SMEM),
        out_shape=jax.ShapeDtypeStruct((8,), jnp.float32),
        scratch_shapes=[pltpu.VMEM((8, P), jnp.float32)],
        compiler_params=pltpu.CompilerParams(
            dimension_semantics=("arbitrary",),
        ),
    )(lam.reshape(1).astype(jnp.float32),
      jnp.swapaxes(conf, 1, 2), jnp.swapaxes(conf_shuffle, 1, 2),
      jnp.swapaxes(conf_interpolation, 1, 2), jnp.swapaxes(loc, 1, 2),
      jnp.swapaxes(loc_shuffle, 1, 2), jnp.swapaxes(loc_interpolation, 1, 2))

    def mmean(s, c):
        return jnp.where(c > 0, s / jnp.maximum(c, 1.0), 0.0)

    interp_loss = mmean(sums[0], sums[1]) / 2.0
    fixmatch = (mmean(sums[2], sums[3]) + mmean(sums[6], sums[3]) / 4.0
                + mmean(sums[4], sums[5]) + mmean(sums[7], sums[5]) / 4.0)
    return interp_loss, fixmatch


def kernel(lam, conf, conf_flip, loc, loc_flip, conf_shuffle,
           conf_interpolation, loc_shuffle, loc_interpolation):
    del conf_flip, loc_flip
    return _isd_loss(lam, conf, conf_shuffle, conf_interpolation, loc,
                     loc_shuffle, loc_interpolation)


# speedup vs baseline: 11.7337x; 2.1834x over previous
import jax
import jax.numpy as jnp
from jax.experimental import pallas as pl
from jax.experimental.pallas import tpu as pltpu

B, P, C = 32, 8732, 21
TP = 1152
NP = 8
HALF = B // 2


def _swap(x):
    return jnp.concatenate([x[:, HALF:, :], x[:, :HALF, :]], axis=1)


def _isd_kernel(lam_ref, conf_ref, confsh_ref, confi_ref, loc_ref, locsh_ref,
                loci_ref, out_ref, acc_ref):
    ip = pl.program_id(0)
    lam = lam_ref[0]

    lanes = jax.lax.broadcasted_iota(jnp.int32, (1, TP), 1)
    valid = (ip * TP + lanes) < P
    validf = valid.astype(jnp.float32)

    tc = conf_ref[...]
    tt = _swap(confsh_ref[...])
    ti = confi_ref[...]

    lmf = (jnp.max(tc, axis=0) > tc[0]).astype(jnp.float32) * validf
    rmf = (jnp.max(tt, axis=0) > tt[0]).astype(jnp.float32) * validf
    inter = lmf * rmf
    only_l = lmf - inter
    only_r = rmf - inter

    mixed = lam * tc + (1.0 - lam) * tt + 1e-07
    interp = ti + 1e-07
    ce = tc + 1e-07
    cte = tt + 1e-07
    log_i = jnp.log(interp)
    log_m = jnp.log(mixed)
    kl_ab = jnp.sum((interp - mixed) * (log_i - log_m), axis=0)
    kl_l = jnp.sum(ce * (jnp.log(ce) - log_i), axis=0)
    kl_r = jnp.sum(cte * (jnp.log(cte) - log_i), axis=0)
    kl_ab = jnp.where(valid, kl_ab, 0.0)
    kl_l = jnp.where(valid, kl_l, 0.0)
    kl_r = jnp.where(valid, kl_r, 0.0)

    tl = loc_ref[...]
    tls = _swap2(locsh_ref[...])
    tli = loci_ref[...]
    se_l = jnp.where(valid, jnp.sum((tli - tl) ** 2, axis=1), 0.0)
    se_r = jnp.where(valid, jnp.sum((tli - tls) ** 2, axis=1), 0.0)

    @pl.when(ip == 0)
    def _init():
        acc_ref[...] = jnp.zeros((8, B, TP), jnp.float32)

    acc_ref[0] += kl_ab * inter
    acc_ref[1] += inter
    acc_ref[2] += kl_l * only_l
    acc_ref[3] += only_l
    acc_ref[4] += kl_r * only_r
    acc_ref[5] += only_r
    acc_ref[6] += se_l * only_l
    acc_ref[7] += se_r * only_r

    @pl.when(ip == NP - 1)
    def _fin():
        for j in range(8):
            out_ref[j] = jnp.sum(acc_ref[j])


def _swap2(x):
    return jnp.concatenate([x[HALF:], x[:HALF]], axis=0)


@jax.jit
def _isd_loss(lam, conf, conf_shuffle, conf_interpolation, loc, loc_shuffle,
              loc_interpolation):
    cspec = pl.BlockSpec((C, B, TP), lambda ip: (0, 0, ip))
    lspec = pl.BlockSpec((B, 4, TP), lambda ip: (0, 0, ip))

    sums = pl.pallas_call(
        _isd_kernel,
        grid=(NP,),
        in_specs=[
            pl.BlockSpec(memory_space=pltpu.SMEM),
            cspec, cspec, cspec, lspec, lspec, lspec,
        ],
        out_specs=pl.BlockSpec(memory_space=pltpu.SMEM),
        out_shape=jax.ShapeDtypeStruct((8,), jnp.float32),
        scratch_shapes=[pltpu.VMEM((8, B, TP), jnp.float32)],
        compiler_params=pltpu.CompilerParams(
            dimension_semantics=("arbitrary",),
        ),
    )(lam.reshape(1).astype(jnp.float32),
      jnp.transpose(conf, (2, 0, 1)), jnp.transpose(conf_shuffle, (2, 0, 1)),
      jnp.transpose(conf_interpolation, (2, 0, 1)),
      jnp.transpose(loc, (0, 2, 1)), jnp.transpose(loc_shuffle, (0, 2, 1)),
      jnp.transpose(loc_interpolation, (0, 2, 1)))

    def mmean(s, c):
        return jnp.where(c > 0, s / jnp.maximum(c, 1.0), 0.0)

    interp_loss = mmean(sums[0], sums[1]) / 2.0
    fixmatch = (mmean(sums[2], sums[3]) + mmean(sums[6], sums[3]) / 4.0
                + mmean(sums[4], sums[5]) + mmean(sums[7], sums[5]) / 4.0)
    return interp_loss, fixmatch


def kernel(lam, conf, conf_flip, loc, loc_flip, conf_shuffle,
           conf_interpolation, loc_shuffle, loc_interpolation):
    del conf_flip, loc_flip
    return _isd_loss(lam, conf, conf_shuffle, conf_interpolation, loc,
                     loc_shuffle, loc_interpolation)


# speedup vs baseline: 15.5772x; 1.3276x over previous
import jax
import jax.numpy as jnp
from jax.experimental import pallas as pl
from jax.experimental.pallas import tpu as pltpu

B, P, C = 32, 8732, 21
TP = 384
NP = 23
HALF = B // 2


def _roll(x):
    return jnp.concatenate([x[HALF:], x[:HALF]], axis=0)


def _isd_kernel(lam_ref, conf_ref, confsh_ref, confi_ref, loc_ref, locsh_ref,
                loci_ref, out_ref, acc_ref):
    ip = pl.program_id(0)
    lam = lam_ref[0]
    q = 1.0 - lam

    lanes = jax.lax.broadcasted_iota(jnp.int32, (1, TP), 1)
    valid = (ip * TP + lanes) < P
    validf = valid.astype(jnp.float32)

    tc0 = conf_ref[0]
    tt0 = _roll(confsh_ref[0])

    s_ab = jnp.zeros((B, TP), jnp.float32)
    s_l = jnp.zeros((B, TP), jnp.float32)
    s_r = jnp.zeros((B, TP), jnp.float32)
    m_l = tc0
    m_r = tt0
    for c in range(C):
        tc_c = tc0 if c == 0 else conf_ref[c]
        tt_c = tt0 if c == 0 else _roll(confsh_ref[c])
        if c > 0:
            m_l = jnp.maximum(m_l, tc_c)
            m_r = jnp.maximum(m_r, tt_c)
        interp = confi_ref[c] + 1e-07
        mixed = lam * tc_c + q * tt_c + 1e-07
        log_i = jnp.log(interp)
        d = log_i - jnp.log(mixed)
        s_ab += (interp - mixed) * d
        ce = tc_c + 1e-07
        cte = tt_c + 1e-07
        s_l += ce * (jnp.log(ce) - log_i)
        s_r += cte * (jnp.log(cte) - log_i)

    lmf = (m_l > tc0).astype(jnp.float32) * validf
    rmf = (m_r > tt0).astype(jnp.float32) * validf
    inter = lmf * rmf
    only_l = lmf - inter
    only_r = rmf - inter

    s_ab = jnp.where(valid, s_ab, 0.0)
    s_l = jnp.where(valid, s_l, 0.0)
    s_r = jnp.where(valid, s_r, 0.0)

    tl = loc_ref[...]
    tls = _roll(locsh_ref[...])
    tli = loci_ref[...]
    se_l = jnp.where(valid, jnp.sum((tli - tl) ** 2, axis=1), 0.0)
    se_r = jnp.where(valid, jnp.sum((tli - tls) ** 2, axis=1), 0.0)

    @pl.when(ip == 0)
    def _init():
        acc_ref[...] = jnp.zeros((8, B, TP), jnp.float32)

    acc_ref[0] += s_ab * inter
    acc_ref[1] += inter
    acc_ref[2] += s_l * only_l
    acc_ref[3] += only_l
    acc_ref[4] += s_r * only_r
    acc_ref[5] += only_r
    acc_ref[6] += se_l * only_l
    acc_ref[7] += se_r * only_r

    @pl.when(ip == NP - 1)
    def _fin():
        for j in range(8):
            out_ref[j] = jnp.sum(acc_ref[j])


@jax.jit
def _isd_loss(lam, conf, conf_shuffle, conf_interpolation, loc, loc_shuffle,
              loc_interpolation):
    cspec = pl.BlockSpec((C, B, TP), lambda ip: (0, 0, ip))
    lspec = pl.BlockSpec((B, 4, TP), lambda ip: (0, 0, ip))

    sums = pl.pallas_call(
        _isd_kernel,
        grid=(NP,),
        in_specs=[
            pl.BlockSpec(memory_space=pltpu.SMEM),
            cspec, cspec, cspec, lspec, lspec, lspec,
        ],
        out_specs=pl.BlockSpec(memory_space=pltpu.SMEM),
        out_shape=jax.ShapeDtypeStruct((8,), jnp.float32),
        scratch_shapes=[pltpu.VMEM((8, B, TP), jnp.float32)],
        compiler_params=pltpu.CompilerParams(
            dimension_semantics=("arbitrary",),
        ),
    )(lam.reshape(1).astype(jnp.float32),
      jnp.transpose(conf, (2, 0, 1)), jnp.transpose(conf_shuffle, (2, 0, 1)),
      jnp.transpose(conf_interpolation, (2, 0, 1)),
      jnp.transpose(loc, (0, 2, 1)), jnp.transpose(loc_shuffle, (0, 2, 1)),
      jnp.transpose(loc_interpolation, (0, 2, 1)))

    def mmean(s, c):
        return jnp.where(c > 0, s / jnp.maximum(c, 1.0), 0.0)

    interp_loss = mmean(sums[0], sums[1]) / 2.0
    fixmatch = (mmean(sums[2], sums[3]) + mmean(sums[6], sums[3]) / 4.0
                + mmean(sums[4], sums[5]) + mmean(sums[7], sums[5]) / 4.0)
    return interp_loss, fixmatch


def kernel(lam, conf, conf_flip, loc, loc_flip, conf_shuffle,
           conf_interpolation, loc_shuffle, loc_interpolation):
    del conf_flip, loc_flip
    return _isd_loss(lam, conf, conf_shuffle, conf_interpolation, loc,
                     loc_shuffle, loc_interpolation)
